# baseline (speedup 1.0000x reference)
import jax
import jax.numpy as jnp
from jax.experimental import pallas as pl
from jax.experimental.pallas import tpu as pltpu

N = 1024
HID = 64
UCHUNK = 1024

_DN_T = (((1,), (1,)), ((), ()))


def _mpnn_kernel(scal_ref, A_ref, E_ref, X_ref, encW_ref, encb_ref,
                 msgW_ref, wecol_ref, msgbcol_ref, Wih_ref, Whh_ref,
                 bih_ref, bhh_ref, ro1W_ref, ro1b_ref, ro2W_ref, ro2b_ref,
                 out_ref, h_ref, hmT_ref, msgs_ref, invd_ref, mask_ref):
    bf16 = jnp.bfloat16
    h0 = jnp.tanh(
        jax.lax.dot_general(X_ref[...], encW_ref[...], _DN_T,
                            preferred_element_type=jnp.float32)
        + encb_ref[...])
    h_ref[...] = h0

    maskf = (A_ref[...] > 0.0).astype(jnp.float32)
    mask_ref[...] = maskf.astype(bf16)
    deg = jnp.sum(maskf, axis=1, keepdims=True)
    invd_ref[...] = jnp.where(deg > 0.0, 1.0 / jnp.maximum(deg, 1.0), 0.0)

    wcol = wecol_ref[...].astype(bf16)
    msgbcol = msgbcol_ref[...]

    def step_body(_, carry):
        h = h_ref[...]
        hmT = jax.lax.dot_general(msgW_ref[:, :HID], h, _DN_T,
                                  preferred_element_type=jnp.float32)
        hmT_ref[...] = (hmT + msgbcol).astype(bf16)
        hmTv = hmT_ref[...]

        def u_body(j, c):
            u0 = j * UCHUNK
            erows = E_ref[pl.ds(u0, UCHUNK), :].astype(bf16)
            marows = mask_ref[pl.ds(u0, UCHUNK), :]
            srows = []
            for i in range(UCHUNK):
                T = jnp.tanh(hmTv + wcol * erows[i:i + 1, :])
                mrow = marows[i:i + 1, :]
                s = jax.lax.dot_general(mrow, T, _DN_T,
                                        preferred_element_type=jnp.float32)
                srows.append(s)
            blk = jnp.concatenate(srows, axis=0)
            msgs_ref[pl.ds(u0, UCHUNK), :] = blk * invd_ref[pl.ds(u0, UCHUNK), :]
            return c

        jax.lax.fori_loop(0, N // UCHUNK, u_body, 0, unroll=False)

        msgs = msgs_ref[...]
        gi = jax.lax.dot_general(msgs, Wih_ref[...], _DN_T,
                                 preferred_element_type=jnp.float32) + bih_ref[...]
        gh = jax.lax.dot_general(h, Whh_ref[...], _DN_T,
                                 preferred_element_type=jnp.float32) + bhh_ref[...]
        r = jax.nn.sigmoid(gi[:, :HID] + gh[:, :HID])
        z = jax.nn.sigmoid(gi[:, HID:2 * HID] + gh[:, HID:2 * HID])
        ng = jnp.tanh(gi[:, 2 * HID:] + r * gh[:, 2 * HID:])
        h_ref[...] = (1.0 - z) * ng + z * h
        return carry

    jax.lax.fori_loop(0, scal_ref[0], step_body, 0)

    hs = h_ref[pl.ds(scal_ref[1], 1), :]
    ht = h_ref[pl.ds(scal_ref[2], 1), :]
    cat = jnp.concatenate([hs, ht], axis=1)
    mid = jax.nn.relu(
        jax.lax.dot(cat, ro1W_ref[...],
                    preferred_element_type=jnp.float32) + ro1b_ref[...])
    out = jax.lax.dot(mid, ro2W_ref[...],
                      preferred_element_type=jnp.float32) + ro2b_ref[...]
    out_ref[...] = jax.nn.sigmoid(out)


def kernel(A, E, X, enc_W, enc_b, msg_W, msg_b, W_ih, W_hh, b_ih, b_hh,
           ro1_W, ro1_b, ro2_W, ro2_b, source, target, steps):
    f32 = jnp.float32
    scal = jnp.stack([jnp.asarray(steps, jnp.int32),
                      jnp.asarray(source, jnp.int32),
                      jnp.asarray(target, jnp.int32)])
    args = (
        scal, A, E, X,
        enc_W, enc_b.reshape(1, HID),
        msg_W, msg_W[:, HID:HID + 1], msg_b.reshape(HID, 1),
        W_ih, W_hh,
        b_ih.reshape(1, 3 * HID), b_hh.reshape(1, 3 * HID),
        ro1_W.T, ro1_b.reshape(1, HID), ro2_W.T, ro2_b.reshape(1, 1),
    )
    in_specs = [pl.BlockSpec(memory_space=pltpu.SMEM)] + \
               [pl.BlockSpec(memory_space=pltpu.VMEM)] * (len(args) - 1)
    out = pl.pallas_call(
        _mpnn_kernel,
        out_shape=jax.ShapeDtypeStruct((1, 1), f32),
        in_specs=in_specs,
        out_specs=pl.BlockSpec(memory_space=pltpu.VMEM),
        scratch_shapes=[
            pltpu.VMEM((N, HID), f32),
            pltpu.VMEM((HID, N), jnp.bfloat16),
            pltpu.VMEM((N, HID), f32),
            pltpu.VMEM((N, 1), f32),
            pltpu.VMEM((N, N), jnp.bfloat16),
        ],
    )(*args)
    return out.reshape(1)


# speedup vs baseline: 1.0266x; 1.0266x over previous
import jax
import jax.numpy as jnp
from jax.experimental import pallas as pl
from jax.experimental.pallas import tpu as pltpu

N = 1024
HID = 64
UCHUNK = 512

_DN_T = (((1,), (1,)), ((), ()))


def _mpnn_kernel(scal_ref, A_ref, E_ref, X_ref, encW_ref, encb_ref,
                 msgW_ref, wecol_ref, msgbcol_ref, Wih_ref, Whh_ref,
                 bih_ref, bhh_ref, ro1W_ref, ro1b_ref, ro2W_ref, ro2b_ref,
                 out_ref, h_ref, hmT_ref, msgs_ref, invd_ref, mask_ref,
                 Ebf_ref):
    bf16 = jnp.bfloat16
    h0 = jnp.tanh(
        jax.lax.dot_general(X_ref[...], encW_ref[...], _DN_T,
                            preferred_element_type=jnp.float32)
        + encb_ref[...])
    h_ref[...] = h0

    maskf = (A_ref[...] > 0.0).astype(jnp.float32)
    mask_ref[...] = maskf.astype(bf16)
    deg = jnp.sum(maskf, axis=1, keepdims=True)
    invd_ref[...] = jnp.where(deg > 0.0, 1.0 / jnp.maximum(deg, 1.0), 0.0)

    Ebf_ref[...] = E_ref[...].astype(bf16)
    wcol = wecol_ref[...].astype(bf16)
    msgbcol = msgbcol_ref[...]

    def step_body(_, carry):
        h = h_ref[...]
        hmT = jax.lax.dot_general(msgW_ref[:, :HID], h, _DN_T,
                                  preferred_element_type=jnp.float32)
        hmT_ref[...] = (hmT + msgbcol).astype(bf16)
        hmTv = hmT_ref[...]

        def u_body(j, c):
            u0 = j * UCHUNK
            erows = Ebf_ref[pl.ds(u0, UCHUNK), :]
            marows = mask_ref[pl.ds(u0, UCHUNK), :]
            srows = []
            for i in range(UCHUNK):
                T = jnp.tanh(hmTv + wcol * erows[i:i + 1, :])
                mrow = marows[i:i + 1, :]
                s = jax.lax.dot_general(mrow, T, _DN_T,
                                        preferred_element_type=jnp.float32)
                srows.append(s)
            blk = jnp.concatenate(srows, axis=0)
            msgs_ref[pl.ds(u0, UCHUNK), :] = blk * invd_ref[pl.ds(u0, UCHUNK), :]
            return c

        jax.lax.fori_loop(0, N // UCHUNK, u_body, 0, unroll=False)

        msgs = msgs_ref[...]
        gi = jax.lax.dot_general(msgs, Wih_ref[...], _DN_T,
                                 preferred_element_type=jnp.float32) + bih_ref[...]
        gh = jax.lax.dot_general(h, Whh_ref[...], _DN_T,
                                 preferred_element_type=jnp.float32) + bhh_ref[...]
        r = jax.nn.sigmoid(gi[:, :HID] + gh[:, :HID])
        z = jax.nn.sigmoid(gi[:, HID:2 * HID] + gh[:, HID:2 * HID])
        ng = jnp.tanh(gi[:, 2 * HID:] + r * gh[:, 2 * HID:])
        h_ref[...] = (1.0 - z) * ng + z * h
        return carry

    jax.lax.fori_loop(0, scal_ref[0], step_body, 0)

    hs = h_ref[pl.ds(scal_ref[1], 1), :]
    ht = h_ref[pl.ds(scal_ref[2], 1), :]
    cat = jnp.concatenate([hs, ht], axis=1)
    mid = jax.nn.relu(
        jax.lax.dot(cat, ro1W_ref[...],
                    preferred_element_type=jnp.float32) + ro1b_ref[...])
    out = jax.lax.dot(mid, ro2W_ref[...],
                      preferred_element_type=jnp.float32) + ro2b_ref[...]
    out_ref[...] = jax.nn.sigmoid(out)


def kernel(A, E, X, enc_W, enc_b, msg_W, msg_b, W_ih, W_hh, b_ih, b_hh,
           ro1_W, ro1_b, ro2_W, ro2_b, source, target, steps):
    f32 = jnp.float32
    scal = jnp.stack([jnp.asarray(steps, jnp.int32),
                      jnp.asarray(source, jnp.int32),
                      jnp.asarray(target, jnp.int32)])
    args = (
        scal, A, E, X,
        enc_W, enc_b.reshape(1, HID),
        msg_W, msg_W[:, HID:HID + 1], msg_b.reshape(HID, 1),
        W_ih, W_hh,
        b_ih.reshape(1, 3 * HID), b_hh.reshape(1, 3 * HID),
        ro1_W.T, ro1_b.reshape(1, HID), ro2_W.T, ro2_b.reshape(1, 1),
    )
    in_specs = [pl.BlockSpec(memory_space=pltpu.SMEM)] + \
               [pl.BlockSpec(memory_space=pltpu.VMEM)] * (len(args) - 1)
    out = pl.pallas_call(
        _mpnn_kernel,
        out_shape=jax.ShapeDtypeStruct((1, 1), f32),
        in_specs=in_specs,
        out_specs=pl.BlockSpec(memory_space=pltpu.VMEM),
        scratch_shapes=[
            pltpu.VMEM((N, HID), f32),
            pltpu.VMEM((HID, N), jnp.bfloat16),
            pltpu.VMEM((N, HID), f32),
            pltpu.VMEM((N, 1), f32),
            pltpu.VMEM((N, N), jnp.bfloat16),
            pltpu.VMEM((N, N), jnp.bfloat16),
        ],
    )(*args)
    return out.reshape(1)
